# baseline (speedup 1.0000x reference)
import functools

import jax
import jax.numpy as jnp
from jax import lax
from jax.experimental import pallas as pl
from jax.experimental.pallas import tpu as pltpu
from jax.experimental.pallas import tpu_sc as plsc

NC = 2
NS = 16
NW = NC * NS
L = 16


def _scores_body(E, D, EPW, CHUNK, NCH,
                 h_hbm, src_hbm, dst_hbm, out_hbm,
                 sidx_v, didx_v, srows_v, drows_v,
                 tmp_a, tmp_b, outall_v,
                 sem_s, sem_d):
    wid = lax.axis_index("s") * NC + lax.axis_index("c")
    base = wid * EPW
    lanes_x16 = lax.iota(jnp.int32, L) * L

    def issue(g, p):
        pltpu.async_copy(h_hbm.at[sidx_v.at[pl.ds(g * CHUNK, CHUNK)]],
                         srows_v.at[p], sem_s.at[p])
        pltpu.async_copy(h_hbm.at[didx_v.at[pl.ds(g * CHUNK, CHUNK)]],
                         drows_v.at[p], sem_d.at[p])

    def wait(g, p):
        pltpu.make_async_copy(h_hbm.at[sidx_v.at[pl.ds(g * CHUNK, CHUNK)]],
                              srows_v.at[p], sem_s.at[p]).wait()
        pltpu.make_async_copy(h_hbm.at[didx_v.at[pl.ds(g * CHUNK, CHUNK)]],
                              drows_v.at[p], sem_d.at[p]).wait()

    pltpu.sync_copy(src_hbm.at[pl.ds(base, EPW)], sidx_v)
    pltpu.sync_copy(dst_hbm.at[pl.ds(base, EPW)], didx_v)

    issue(0, 0)

    def chunk_body(g, carry):
        p = lax.rem(g, 2)
        wait(g, p)
        issue(lax.rem(g + 1, NCH), 1 - p)
        srows = srows_v.at[p]
        drows = drows_v.at[p]
        for gi in range(CHUNK // L):
            tmp_v = tmp_a if gi % 2 == 0 else tmp_b
            jb = gi * L
            for jj in range(L):
                e = jb + jj
                a = srows[e, pl.ds(0, L)] * drows[e, pl.ds(0, L)]
                for cc in range(1, D // L):
                    a = a + (srows[e, pl.ds(cc * L, L)]
                             * drows[e, pl.ds(cc * L, L)])
                tmp_v[pl.ds(jj * L, L)] = a
            r = plsc.load_gather(tmp_v, [lanes_x16])
            for l in range(1, L):
                r = r + plsc.load_gather(tmp_v, [lanes_x16 + l])
            r = 1.0 / (1.0 + jnp.exp(-r))
            outall_v[pl.ds(g * CHUNK + jb, L)] = r
        return carry

    lax.fori_loop(0, NCH, chunk_body, 0, unroll=False)
    wait(0, lax.rem(jnp.int32(NCH), 2))

    pltpu.sync_copy(outall_v, out_hbm.at[pl.ds(base, EPW)])


def kernel(h, edge_index):
    N, D = h.shape
    E = edge_index.shape[1]
    EPW = E // NW
    CHUNK = 80
    NCH = EPW // CHUNK
    assert EPW * NW == E and NCH * CHUNK == EPW and D % L == 0

    src = edge_index[0]
    dst = edge_index[1]

    mesh = plsc.VectorSubcoreMesh(core_axis_name="c", subcore_axis_name="s",
                                  num_cores=NC, num_subcores=NS)
    body = functools.partial(_scores_body, E, D, EPW, CHUNK, NCH)
    f = pl.kernel(
        body,
        out_type=jax.ShapeDtypeStruct((E,), jnp.float32),
        mesh=mesh,
        compiler_params=pltpu.CompilerParams(needs_layout_passes=False),
        scratch_types=[
            pltpu.VMEM((EPW,), jnp.int32),
            pltpu.VMEM((EPW,), jnp.int32),
            pltpu.VMEM((2, CHUNK, D), jnp.float32),
            pltpu.VMEM((2, CHUNK, D), jnp.float32),
            pltpu.VMEM((L * L,), jnp.float32),
            pltpu.VMEM((L * L,), jnp.float32),
            pltpu.VMEM((EPW,), jnp.float32),
            pltpu.SemaphoreType.DMA((2,)),
            pltpu.SemaphoreType.DMA((2,)),
        ],
    )
    return f(h, src, dst)
